# baseline (speedup 1.0000x reference)
import functools
import math

import jax
import jax.numpy as jnp
import numpy as np
from jax import lax
from jax.experimental import pallas as pl
from jax.experimental.pallas import tpu as pltpu
from jax.experimental.pallas import tpu_sc as plsc

NUM_EMB = 1000000
D = 64
B = 4096
L = 200
ROWS = B * L
NC = 2
NS = 16
NW = NC * NS
SEQ_PER_W = B // NW
HALF = L // 2


def _pos_encoding() -> np.ndarray:
    position = np.arange(L, dtype=np.float32)[:, None]
    div_term = np.exp(np.arange(0, D, 2, dtype=np.float32) * (-math.log(10000.0) / D))
    pe = np.zeros((L, D), dtype=np.float32)
    pe[:, 0::2] = np.sin(position * div_term)
    pe[:, 1::2] = np.cos(position * div_term)
    return pe.reshape(HALF, 2 * D)


_PE2 = _pos_encoding()

_mesh = plsc.VectorSubcoreMesh(core_axis_name="c", subcore_axis_name="s")


@functools.partial(
    pl.kernel,
    mesh=_mesh,
    out_type=jax.ShapeDtypeStruct((B, L, D), jnp.float32),
    scratch_types=[
        pltpu.VMEM((2, L, 2 * D), jnp.float32),
        pltpu.VMEM((2, L, D), jnp.float32),
        pltpu.VMEM((HALF, 2 * D), jnp.float32),
        pltpu.VMEM((2 * L,), jnp.int32),
        pltpu.SemaphoreType.DMA((2,)),
        pltpu.SemaphoreType.DMA((2,)),
    ],
)
def _embed_pe(idx_hbm, w2_hbm, pe_hbm, out_hbm,
              bufg_v, bufs_v, pe_v, idx_v, gsem, ssem):
    wid = lax.axis_index("s") * NC + lax.axis_index("c")
    seq0 = wid * SEQ_PER_W
    pltpu.sync_copy(pe_hbm, pe_v)

    def issue(c, slot):
        pltpu.sync_copy(idx_hbm.at[pl.ds((seq0 + c) * L, L)], idx_v.at[pl.ds(slot * L, L)])
        pltpu.async_copy(w2_hbm.at[idx_v.at[pl.ds(slot * L, L)]], bufg_v.at[slot], gsem.at[slot])

    issue(0, 0)

    def chunk_body(c, carry):
        slot = lax.rem(c, 2)
        nxt = 1 - slot
        @pl.when(c + 1 < SEQ_PER_W)
        def _():
            issue(c + 1, nxt)

        pltpu.make_async_copy(
            w2_hbm.at[idx_v.at[pl.ds(slot * L, L)]], bufg_v.at[slot], gsem.at[slot]
        ).wait()

        @pl.when(c >= 2)
        def _():
            pltpu.make_async_copy(
                bufs_v.at[pl.ds(slot, 1)], out_hbm.at[pl.ds(seq0 + c - 2, 1)], ssem.at[slot]
            ).wait()

        def row_body(p, carry2):
            i0 = 2 * p
            i1 = i0 + 1
            for v in range(4):
                sl = pl.ds(v * 16, 16)
                bufs_v[slot, i0, sl] = bufg_v[slot, i0, sl] + pe_v[p, sl]
            for v in range(4):
                sl = pl.ds(v * 16, 16)
                bufs_v[slot, i1, sl] = (
                    bufg_v[slot, i1, sl] + pe_v[p, pl.ds(D + v * 16, 16)]
                )
            return carry2

        lax.fori_loop(0, HALF, row_body, 0)
        pltpu.async_copy(
            bufs_v.at[pl.ds(slot, 1)], out_hbm.at[pl.ds(seq0 + c, 1)], ssem.at[slot]
        )
        return carry

    lax.fori_loop(0, SEQ_PER_W, chunk_body, 0)
    pltpu.make_async_copy(
        bufs_v.at[pl.ds(0, 1)], out_hbm.at[pl.ds(seq0 + SEQ_PER_W - 2, 1)], ssem.at[0]
    ).wait()
    pltpu.make_async_copy(
        bufs_v.at[pl.ds(1, 1)], out_hbm.at[pl.ds(seq0 + SEQ_PER_W - 1, 1)], ssem.at[1]
    ).wait()


def kernel(X, W):
    idx = X.reshape(ROWS).astype(jnp.int32)
    w2 = jnp.pad(W, ((0, 0), (0, D)))
    pe = jnp.asarray(_PE2)
    return _embed_pe(idx, w2, pe)


# speedup vs baseline: 1.2922x; 1.2922x over previous
import functools
import math

import jax
import jax.numpy as jnp
import numpy as np
from jax import lax
from jax.experimental import pallas as pl
from jax.experimental.pallas import tpu as pltpu
from jax.experimental.pallas import tpu_sc as plsc

NUM_EMB = 1000000
D = 64
B = 4096
L = 200
ROWS = B * L
NC = 2
NS = 16
NW = NC * NS
SEQ_PER_W = B // NW
HALF = L // 2


def _pos_encoding() -> np.ndarray:
    position = np.arange(L, dtype=np.float32)[:, None]
    div_term = np.exp(np.arange(0, D, 2, dtype=np.float32) * (-math.log(10000.0) / D))
    pe = np.zeros((L, D), dtype=np.float32)
    pe[:, 0::2] = np.sin(position * div_term)
    pe[:, 1::2] = np.cos(position * div_term)
    return pe.reshape(HALF, 2 * D)


_PE2 = _pos_encoding()

_mesh = plsc.VectorSubcoreMesh(core_axis_name="c", subcore_axis_name="s")


@functools.partial(
    pl.kernel,
    mesh=_mesh,
    out_type=jax.ShapeDtypeStruct((B, L, D), jnp.float32),
    scratch_types=[
        pltpu.VMEM((2, L, 2 * D), jnp.float32),
        pltpu.VMEM((2, L, D), jnp.float32),
        pltpu.VMEM((HALF, 2 * D), jnp.float32),
        pltpu.VMEM((2 * L,), jnp.int32),
        pltpu.SemaphoreType.DMA((2,)),
        pltpu.SemaphoreType.DMA((2,)),
    ],
)
def _embed_pe(idx_hbm, w2_hbm, pe_hbm, out_hbm,
              bufg_v, bufs_v, pe_v, idx_v, gsem, ssem):
    wid = lax.axis_index("s") * NC + lax.axis_index("c")
    seq0 = wid * SEQ_PER_W
    pltpu.sync_copy(pe_hbm, pe_v)

    def issue(c, slot):
        pltpu.sync_copy(idx_hbm.at[pl.ds((seq0 + c) * L, L)],
                        idx_v.at[pl.ds(slot * L, L)])
        pltpu.async_copy(w2_hbm.at[idx_v.at[pl.ds(slot * L, L)]],
                         bufg_v.at[slot], gsem.at[slot])

    issue(0, 0)

    def chunk(c, slot, nxt):
        @pl.when(c + 1 < SEQ_PER_W)
        def _():
            issue(c + 1, nxt)

        pltpu.make_async_copy(
            w2_hbm.at[idx_v.at[pl.ds(slot * L, L)]], bufg_v.at[slot],
            gsem.at[slot],
        ).wait()

        @pl.when(c >= 2)
        def _():
            pltpu.make_async_copy(
                bufs_v.at[pl.ds(slot, 1)], out_hbm.at[pl.ds(seq0 + c - 2, 1)],
                ssem.at[slot],
            ).wait()

        @plsc.parallel_loop(0, HALF, unroll=2)
        def row_body(p):
            i0 = 2 * p
            i1 = i0 + 1
            for v in range(4):
                sl = pl.ds(v * 16, 16)
                bufs_v[slot, i0, sl] = bufg_v[slot, i0, sl] + pe_v[p, sl]
            for v in range(4):
                sl = pl.ds(v * 16, 16)
                bufs_v[slot, i1, sl] = (
                    bufg_v[slot, i1, sl] + pe_v[p, pl.ds(D + v * 16, 16)]
                )

        pltpu.async_copy(
            bufs_v.at[pl.ds(slot, 1)], out_hbm.at[pl.ds(seq0 + c, 1)],
            ssem.at[slot],
        )

    def super_body(t, carry):
        c0 = 2 * t
        chunk(c0, 0, 1)
        chunk(c0 + 1, 1, 0)
        return carry

    lax.fori_loop(0, SEQ_PER_W // 2, super_body, 0)
    pltpu.make_async_copy(
        bufs_v.at[pl.ds(0, 1)], out_hbm.at[pl.ds(seq0 + SEQ_PER_W - 2, 1)],
        ssem.at[0],
    ).wait()
    pltpu.make_async_copy(
        bufs_v.at[pl.ds(1, 1)], out_hbm.at[pl.ds(seq0 + SEQ_PER_W - 1, 1)],
        ssem.at[1],
    ).wait()


def kernel(X, W):
    idx = X.reshape(ROWS).astype(jnp.int32)
    w2 = jnp.pad(W, ((0, 0), (0, D)))
    pe = jnp.asarray(_PE2)
    return _embed_pe(idx, w2, pe)
